# baseline (speedup 1.0000x reference)
import functools

import jax
import jax.numpy as jnp
from jax import lax
from jax.experimental import pallas as pl
from jax.experimental.pallas import tpu as pltpu
from jax.experimental.pallas import tpu_sc as plsc

N = 10000
E = 320000
F = 128
H = 128
C = 40
CP = 48

NC = 2
NS = 16

B = 128
CHUNKS = 160
EPT = CHUNKS * B
E_PAD = NS * EPT
NPAD = 10112
RPT = NPAD // NS

_MESH = plsc.VectorSubcoreMesh(
    core_axis_name="c", subcore_axis_name="s", num_cores=NC, num_subcores=NS
)
_SC_PARAMS = pltpu.CompilerParams(use_tc_tiling_on_sc=False)


DEG_CHUNKS = CHUNKS // NC


@functools.partial(
    pl.kernel,
    out_type=jax.ShapeDtypeStruct((NC * NPAD,), jnp.float32),
    mesh=_MESH,
    compiler_params=_SC_PARAMS,
    scratch_types=[
        pltpu.VMEM((DEG_CHUNKS, B), jnp.int32),
        pltpu.VMEM((B,), jnp.float32),
        pltpu.VMEM((RPT,), jnp.float32),
        pltpu.VMEM_SHARED((NPAD,), jnp.float32),
        pltpu.SemaphoreType.DMA,
    ],
)
def _sc_degree(dst_hbm, out_hbm, dstb, onesv, zb, acc, dsem):
    c = lax.axis_index("c")
    s = lax.axis_index("s")

    pltpu.sync_copy(dst_hbm.at[s, pl.ds(c * DEG_CHUNKS, DEG_CHUNKS)], dstb)
    for j in range(RPT // 16):
        zb[pl.ds(j * 16, 16)] = jnp.zeros((16,), jnp.float32)
    for j in range(B // 16):
        onesv[pl.ds(j * 16, 16)] = jnp.ones((16,), jnp.float32)

    pltpu.sync_copy(zb, acc.at[pl.ds(s * RPT, RPT)])
    plsc.subcore_barrier()

    def body(i, carry):
        pltpu.async_copy(onesv, acc.at[dstb.at[i]], dsem, add=True)
        return carry

    lax.fori_loop(0, DEG_CHUNKS, body, 0)

    def drain(i, carry):
        pltpu.make_async_copy(onesv, acc.at[dstb.at[i]], dsem).wait()
        return carry

    lax.fori_loop(0, DEG_CHUNKS, drain, 0)
    plsc.subcore_barrier()
    pltpu.sync_copy(acc.at[pl.ds(s * RPT, RPT)], zb)
    pltpu.sync_copy(zb, out_hbm.at[pl.ds(c * NPAD + s * RPT, RPT)])


NBUF = 4
PFD = 2
OUTCH = 4


def _make_prop(fwh):
    @functools.partial(
        pl.kernel,
        out_type=jax.ShapeDtypeStruct((NC, NPAD, fwh), jnp.float32),
        mesh=_MESH,
        compiler_params=_SC_PARAMS,
        scratch_types=[
            pltpu.VMEM((CHUNKS, B), jnp.int32),
            pltpu.VMEM((CHUNKS, B), jnp.int32),
            [pltpu.VMEM((B, fwh), jnp.float32) for _ in range(NBUF)],
            pltpu.VMEM((RPT // OUTCH, fwh), jnp.float32),
            pltpu.VMEM_SHARED((NPAD, fwh), jnp.float32),
            [pltpu.SemaphoreType.DMA for _ in range(NBUF)],
            [pltpu.SemaphoreType.DMA for _ in range(NBUF)],
            pltpu.SemaphoreType.DMA,
        ],
    )
    def _prop(src_hbm, dst_hbm, g_hbm, out_hbm, srcb, dstb, rows, stg, acc,
              gsem, ssem, zsem):
        c = lax.axis_index("c")
        s = lax.axis_index("s")

        pltpu.sync_copy(src_hbm.at[s], srcb)
        pltpu.sync_copy(dst_hbm.at[s], dstb)

        for r in range(8):
            for j in range(fwh // 16):
                stg[r, pl.ds(j * 16, 16)] = jnp.zeros((16,), jnp.float32)

        def zinit(i, carry):
            pltpu.async_copy(
                stg.at[pl.ds(0, 8)], acc.at[pl.ds(s * RPT + i * 8, 8)], zsem
            )
            return carry

        lax.fori_loop(0, RPT // 8, zinit, 0)
        pltpu.make_async_copy(
            out_hbm.at[c, pl.ds(s * RPT, RPT)], acc.at[pl.ds(s * RPT, RPT)], zsem
        ).wait()
        plsc.subcore_barrier()

        gtab = g_hbm.at[c]

        def _gather_start(i, k):
            pltpu.async_copy(gtab.at[srcb.at[i]], rows[k], gsem[k])

        def _gather_wait(i, k):
            pltpu.make_async_copy(gtab.at[srcb.at[i]], rows[k], gsem[k]).wait()

        def _scat_start(i, k):
            pltpu.async_copy(rows[k], acc.at[dstb.at[i]], ssem[k], add=True)

        def _scat_wait(i, k):
            pltpu.make_async_copy(rows[k], acc.at[dstb.at[i]], ssem[k]).wait()

        for k in range(PFD):
            _gather_start(k, k)

        def body(jj, carry):
            for k in range(NBUF):
                i = jj * NBUF + k
                _gather_wait(i, k)
                _scat_start(i, k)
                kp = (k + PFD) % NBUF
                ip = i + PFD
                if k + PFD < NBUF:
                    @pl.when(jj > 0)
                    def _():
                        _scat_wait(ip - NBUF, kp)
                        _gather_start(ip, kp)

                    @pl.when(jj == 0)
                    def _():
                        _gather_start(ip, kp)
                else:

                    @pl.when(ip < CHUNKS)
                    def _():
                        _scat_wait(ip - NBUF, kp)
                        _gather_start(ip, kp)

            return carry

        lax.fori_loop(0, CHUNKS // NBUF, body, 0)
        for k in range(NBUF):
            i = CHUNKS - NBUF + k
            _scat_wait(i, k)

        plsc.subcore_barrier()
        ocr = RPT // OUTCH
        for j in range(OUTCH):
            r0 = s * RPT + j * ocr
            pltpu.sync_copy(acc.at[pl.ds(r0, ocr)], stg)
            pltpu.sync_copy(stg, out_hbm.at[c, pl.ds(r0, ocr)])

    return _prop


_prop_h = _make_prop(H // 2)
_prop_c = _make_prop(CP // 2)


RB = 2000
GRID = N // RB


def _tc1_body(degt_ref, x_ref, w_ref, dinv_ref, g_ref):
    deg = degt_ref[:, 0:1] + degt_ref[:, 1:2] + 1.0
    dinv = lax.rsqrt(deg)
    dinv_ref[...] = dinv
    z = dinv * jnp.dot(x_ref[...], w_ref[...], preferred_element_type=jnp.float32)
    g_ref[0] = z[:, : H // 2]
    g_ref[1] = z[:, H // 2 :]


def _tc1(degt, x, w1):
    return pl.pallas_call(
        _tc1_body,
        grid=(GRID,),
        in_specs=[
            pl.BlockSpec((RB, 2), lambda i: (i, 0)),
            pl.BlockSpec((RB, F), lambda i: (i, 0)),
            pl.BlockSpec((F, H), lambda i: (0, 0)),
        ],
        out_specs=[
            pl.BlockSpec((RB, 1), lambda i: (i, 0)),
            pl.BlockSpec((NC, RB, H // 2), lambda i: (0, i, 0)),
        ],
        out_shape=[
            jax.ShapeDtypeStruct((N, 1), jnp.float32),
            jax.ShapeDtypeStruct((NC, N, H // 2), jnp.float32),
        ],
    )(degt, x, w1)


def _tc_mid_body(s_ref, g_ref, dinv_ref, b_ref, w_ref, out_ref):
    fo2 = out_ref.shape[2]
    dinv = dinv_ref[...]
    sg = jnp.concatenate([s_ref[0] + g_ref[0], s_ref[1] + g_ref[1]], axis=1)
    h = jnp.maximum(dinv * sg + b_ref[...], 0.0)
    z = dinv * jnp.dot(h, w_ref[...], preferred_element_type=jnp.float32)
    out_ref[0] = z[:, :fo2]
    out_ref[1] = z[:, fo2:]


def _tc_mid(s, g, dinv, b, w, fout):
    fin = 2 * g.shape[2]
    return pl.pallas_call(
        _tc_mid_body,
        grid=(GRID,),
        in_specs=[
            pl.BlockSpec((NC, RB, fin // 2), lambda i: (0, i, 0)),
            pl.BlockSpec((NC, RB, fin // 2), lambda i: (0, i, 0)),
            pl.BlockSpec((RB, 1), lambda i: (i, 0)),
            pl.BlockSpec((1, fin), lambda i: (0, 0)),
            pl.BlockSpec((fin, fout), lambda i: (0, 0)),
        ],
        out_specs=pl.BlockSpec((NC, RB, fout // 2), lambda i: (0, i, 0)),
        out_shape=jax.ShapeDtypeStruct((NC, N, fout // 2), jnp.float32),
    )(s, g, dinv, b, w)


def _tc_out_body(s_ref, g_ref, dinv_ref, b_ref, out_ref):
    sg = jnp.concatenate([s_ref[0] + g_ref[0], s_ref[1] + g_ref[1]], axis=1)
    o = dinv_ref[...] * sg + b_ref[...]
    col = lax.broadcasted_iota(jnp.int32, (RB, CP), 1)
    o = jnp.where(col < C, o, -jnp.inf)
    m = jnp.max(o, axis=1, keepdims=True)
    e = o - m
    lse = jnp.log(jnp.sum(jnp.exp(e), axis=1, keepdims=True))
    out_ref[...] = e - lse


def _tc_out(s, g, dinv, b):
    return pl.pallas_call(
        _tc_out_body,
        grid=(GRID,),
        in_specs=[
            pl.BlockSpec((NC, RB, CP // 2), lambda i: (0, i, 0)),
            pl.BlockSpec((NC, RB, CP // 2), lambda i: (0, i, 0)),
            pl.BlockSpec((RB, 1), lambda i: (i, 0)),
            pl.BlockSpec((1, CP), lambda i: (0, 0)),
        ],
        out_specs=pl.BlockSpec((RB, CP), lambda i: (i, 0)),
        out_shape=jax.ShapeDtypeStruct((N, CP), jnp.float32),
    )(s, g, dinv, b)


def kernel(x, edge_index, W1, b1, W2, b2, W3, b3):
    src = edge_index[0]
    dst = edge_index[1]
    pad_e = E_PAD - E
    srcp = jnp.concatenate([src, jnp.zeros((pad_e,), src.dtype)])
    dstp = jnp.concatenate([dst, jnp.full((pad_e,), N, dst.dtype)])
    src3 = srcp.reshape(NS, CHUNKS, B)
    dst3 = dstp.reshape(NS, CHUNKS, B)

    deg = _sc_degree(dst3).reshape(NC, NPAD)
    degt = jnp.transpose(deg)[:N]

    dinv, g1 = _tc1(degt, x, W1)
    s1 = _prop_h(src3, dst3, g1)
    g2 = _tc_mid(s1, g1, dinv, b1.reshape(1, H), W2, H)
    s2 = _prop_h(src3, dst3, g2)
    w3p = jnp.pad(W3, ((0, 0), (0, CP - C)))
    b3p = jnp.pad(b3, (0, CP - C))
    g3 = _tc_mid(s2, g2, dinv, b2.reshape(1, H), w3p, CP)
    s3 = _prop_c(src3, dst3, g3)
    out = _tc_out(s3, g3, dinv, b3p.reshape(1, CP))
    return out[:, :C]


# speedup vs baseline: 1.7038x; 1.7038x over previous
import functools

import jax
import jax.numpy as jnp
from jax import lax
from jax.experimental import pallas as pl
from jax.experimental.pallas import tpu as pltpu
from jax.experimental.pallas import tpu_sc as plsc

N = 10000
E = 320000
F = 128
H = 128
C = 40
CP = 48

NC = 2
NS = 16

B = 128
CHUNKS = 160
EPT = CHUNKS * B
E_PAD = NS * EPT
NPAD = 10112
RPT = NPAD // NS

_MESH = plsc.VectorSubcoreMesh(
    core_axis_name="c", subcore_axis_name="s", num_cores=NC, num_subcores=NS
)
_SC_PARAMS = pltpu.CompilerParams(use_tc_tiling_on_sc=False)


DEG_CHUNKS = CHUNKS // NC


@functools.partial(
    pl.kernel,
    out_type=jax.ShapeDtypeStruct((NC * NPAD,), jnp.float32),
    mesh=_MESH,
    compiler_params=_SC_PARAMS,
    scratch_types=[
        pltpu.VMEM((DEG_CHUNKS, B), jnp.int32),
        pltpu.VMEM((B,), jnp.float32),
        pltpu.VMEM((640,), jnp.float32),
        pltpu.VMEM_SHARED((NPAD,), jnp.float32),
        pltpu.SemaphoreType.DMA,
    ],
)
def _sc_degree(dst_hbm, out_hbm, dstb, onesv, zb, acc, dsem):
    c = lax.axis_index("c")
    s = lax.axis_index("s")

    pltpu.sync_copy(dst_hbm.at[s, pl.ds(c * DEG_CHUNKS, DEG_CHUNKS)], dstb)
    for j in range(640 // 16):
        zb[pl.ds(j * 16, 16)] = jnp.zeros((16,), jnp.float32)
    for j in range(B // 16):
        onesv[pl.ds(j * 16, 16)] = jnp.ones((16,), jnp.float32)

    pltpu.sync_copy(zb.at[pl.ds(0, RPT)], acc.at[pl.ds(s * RPT, RPT)])
    plsc.subcore_barrier()

    def body(i, carry):
        pltpu.async_copy(onesv, acc.at[dstb.at[i]], dsem, add=True)
        return carry

    lax.fori_loop(0, DEG_CHUNKS, body, 0)

    def drain(i, carry):
        pltpu.make_async_copy(onesv, acc.at[dstb.at[i]], dsem).wait()
        return carry

    lax.fori_loop(0, DEG_CHUNKS, drain, 0)
    plsc.subcore_barrier()
    pltpu.sync_copy(acc.at[pl.ds(s * RPT, RPT)], zb.at[pl.ds(0, RPT)])
    pltpu.sync_copy(zb.at[pl.ds(0, RPT)], out_hbm.at[pl.ds(c * NPAD + s * RPT, RPT)])


NBUF = 4
PFD = 2
OUTCH = 4


def _make_prop(fwh):
    @functools.partial(
        pl.kernel,
        out_type=jax.ShapeDtypeStruct((NC, NPAD, fwh), jnp.float32),
        mesh=_MESH,
        compiler_params=_SC_PARAMS,
        scratch_types=[
            pltpu.VMEM((CHUNKS, B), jnp.int32),
            pltpu.VMEM((CHUNKS, B), jnp.int32),
            [pltpu.VMEM((B, fwh), jnp.float32) for _ in range(NBUF)],
            pltpu.VMEM((RPT // OUTCH, fwh), jnp.float32),
            pltpu.VMEM_SHARED((NPAD, fwh), jnp.float32),
            [pltpu.SemaphoreType.DMA for _ in range(NBUF)],
            [pltpu.SemaphoreType.DMA for _ in range(NBUF)],
            pltpu.SemaphoreType.DMA,
        ],
    )
    def _prop(src_hbm, dst_hbm, g_hbm, out_hbm, srcb, dstb, rows, stg, acc,
              gsem, ssem, zsem):
        c = lax.axis_index("c")
        s = lax.axis_index("s")

        pltpu.sync_copy(src_hbm.at[s], srcb)
        pltpu.sync_copy(dst_hbm.at[s], dstb)

        for r in range(8):
            for j in range(fwh // 16):
                stg[r, pl.ds(j * 16, 16)] = jnp.zeros((16,), jnp.float32)

        def zinit(i, carry):
            pltpu.async_copy(
                stg.at[pl.ds(0, 8)], acc.at[pl.ds(s * RPT + i * 8, 8)], zsem
            )
            return carry

        lax.fori_loop(0, RPT // 8, zinit, 0)
        pltpu.make_async_copy(
            out_hbm.at[c, pl.ds(s * RPT, RPT)], acc.at[pl.ds(s * RPT, RPT)], zsem
        ).wait()
        plsc.subcore_barrier()

        gtab = g_hbm.at[c]

        def _gather_start(i, k):
            pltpu.async_copy(gtab.at[srcb.at[i]], rows[k], gsem[k])

        def _gather_wait(i, k):
            pltpu.make_async_copy(gtab.at[srcb.at[i]], rows[k], gsem[k]).wait()

        def _scat_start(i, k):
            pltpu.async_copy(rows[k], acc.at[dstb.at[i]], ssem[k], add=True)

        def _scat_wait(i, k):
            pltpu.make_async_copy(rows[k], acc.at[dstb.at[i]], ssem[k]).wait()

        for k in range(PFD):
            _gather_start(k, k)

        def body(jj, carry):
            for k in range(NBUF):
                i = jj * NBUF + k
                _gather_wait(i, k)
                _scat_start(i, k)
                kp = (k + PFD) % NBUF
                ip = i + PFD
                if k + PFD < NBUF:
                    @pl.when(jj > 0)
                    def _():
                        _scat_wait(ip - NBUF, kp)
                        _gather_start(ip, kp)

                    @pl.when(jj == 0)
                    def _():
                        _gather_start(ip, kp)
                else:

                    @pl.when(ip < CHUNKS)
                    def _():
                        _scat_wait(ip - NBUF, kp)
                        _gather_start(ip, kp)

            return carry

        lax.fori_loop(0, CHUNKS // NBUF, body, 0)
        for k in range(NBUF):
            i = CHUNKS - NBUF + k
            _scat_wait(i, k)

        plsc.subcore_barrier()
        ocr = RPT // OUTCH
        for j in range(OUTCH):
            r0 = s * RPT + j * ocr
            pltpu.sync_copy(acc.at[pl.ds(r0, ocr)], stg)
            pltpu.sync_copy(stg, out_hbm.at[c, pl.ds(r0, ocr)])

    return _prop


_prop_h = _make_prop(H // 2)
_prop_c = _make_prop(CP // 2)


RB = 2000
GRID = N // RB


def _tc1_body(degt_ref, x_ref, w_ref, dinv_ref, g_ref):
    deg = degt_ref[:, 0:1] + degt_ref[:, 1:2] + 1.0
    dinv = lax.rsqrt(deg)
    dinv_ref[...] = dinv
    z = dinv * jnp.dot(x_ref[...], w_ref[...], preferred_element_type=jnp.float32)
    g_ref[0] = z[:, : H // 2]
    g_ref[1] = z[:, H // 2 :]


def _tc1(degt, x, w1):
    return pl.pallas_call(
        _tc1_body,
        grid=(GRID,),
        in_specs=[
            pl.BlockSpec((RB, 2), lambda i: (i, 0)),
            pl.BlockSpec((RB, F), lambda i: (i, 0)),
            pl.BlockSpec((F, H), lambda i: (0, 0)),
        ],
        out_specs=[
            pl.BlockSpec((RB, 1), lambda i: (i, 0)),
            pl.BlockSpec((NC, RB, H // 2), lambda i: (0, i, 0)),
        ],
        out_shape=[
            jax.ShapeDtypeStruct((N, 1), jnp.float32),
            jax.ShapeDtypeStruct((NC, N, H // 2), jnp.float32),
        ],
    )(degt, x, w1)


def _tc_mid_body(s_ref, g_ref, dinv_ref, b_ref, w_ref, out_ref):
    fo2 = out_ref.shape[2]
    dinv = dinv_ref[...]
    sg = jnp.concatenate([s_ref[0] + g_ref[0], s_ref[1] + g_ref[1]], axis=1)
    h = jnp.maximum(dinv * sg + b_ref[...], 0.0)
    z = dinv * jnp.dot(h, w_ref[...], preferred_element_type=jnp.float32)
    out_ref[0] = z[:, :fo2]
    out_ref[1] = z[:, fo2:]


def _tc_mid(s, g, dinv, b, w, fout):
    fin = 2 * g.shape[2]
    return pl.pallas_call(
        _tc_mid_body,
        grid=(GRID,),
        in_specs=[
            pl.BlockSpec((NC, RB, fin // 2), lambda i: (0, i, 0)),
            pl.BlockSpec((NC, RB, fin // 2), lambda i: (0, i, 0)),
            pl.BlockSpec((RB, 1), lambda i: (i, 0)),
            pl.BlockSpec((1, fin), lambda i: (0, 0)),
            pl.BlockSpec((fin, fout), lambda i: (0, 0)),
        ],
        out_specs=pl.BlockSpec((NC, RB, fout // 2), lambda i: (0, i, 0)),
        out_shape=jax.ShapeDtypeStruct((NC, N, fout // 2), jnp.float32),
    )(s, g, dinv, b, w)


def _tc_out_body(s_ref, g_ref, dinv_ref, b_ref, out_ref):
    sg = jnp.concatenate([s_ref[0] + g_ref[0], s_ref[1] + g_ref[1]], axis=1)
    o = dinv_ref[...] * sg + b_ref[...]
    col = lax.broadcasted_iota(jnp.int32, (RB, CP), 1)
    o = jnp.where(col < C, o, -jnp.inf)
    m = jnp.max(o, axis=1, keepdims=True)
    e = o - m
    lse = jnp.log(jnp.sum(jnp.exp(e), axis=1, keepdims=True))
    out_ref[...] = e - lse


def _tc_out(s, g, dinv, b):
    return pl.pallas_call(
        _tc_out_body,
        grid=(GRID,),
        in_specs=[
            pl.BlockSpec((NC, RB, CP // 2), lambda i: (0, i, 0)),
            pl.BlockSpec((NC, RB, CP // 2), lambda i: (0, i, 0)),
            pl.BlockSpec((RB, 1), lambda i: (i, 0)),
            pl.BlockSpec((1, CP), lambda i: (0, 0)),
        ],
        out_specs=pl.BlockSpec((RB, CP), lambda i: (i, 0)),
        out_shape=jax.ShapeDtypeStruct((N, CP), jnp.float32),
    )(s, g, dinv, b)


def kernel(x, edge_index, W1, b1, W2, b2, W3, b3):
    src = edge_index[0]
    dst = edge_index[1]
    pad_e = E_PAD - E
    pidx = jnp.arange(pad_e, dtype=src.dtype)
    srcp = jnp.concatenate([src, pidx % N])
    dstp = jnp.concatenate([dst, N + pidx % (NPAD - N)])
    src3 = srcp.reshape(NS, CHUNKS, B)
    dst3 = dstp.reshape(NS, CHUNKS, B)

    deg = _sc_degree(dst3).reshape(NC, NPAD)
    degt = jnp.transpose(deg)[:N]

    dinv, g1 = _tc1(degt, x, W1)
    s1 = _prop_h(src3, dst3, g1)
    g2 = _tc_mid(s1, g1, dinv, b1.reshape(1, H), W2, H)
    s2 = _prop_h(src3, dst3, g2)
    w3p = jnp.pad(W3, ((0, 0), (0, CP - C)))
    b3p = jnp.pad(b3, (0, CP - C))
    g3 = _tc_mid(s2, g2, dinv, b2.reshape(1, H), w3p, CP)
    s3 = _prop_c(src3, dst3, g3)
    out = _tc_out(s3, g3, dinv, b3p.reshape(1, CP))
    return out[:, :C]


# speedup vs baseline: 1.8759x; 1.1010x over previous
import functools

import jax
import jax.numpy as jnp
from jax import lax
from jax.experimental import pallas as pl
from jax.experimental.pallas import tpu as pltpu
from jax.experimental.pallas import tpu_sc as plsc

N = 10000
E = 320000
F = 128
H = 128
C = 40
CP = 48

NC = 2
NS = 16

B = 80
CHUNKS = 250
EPT = CHUNKS * B
NPAD = 10112
RPT = NPAD // NS

_MESH = plsc.VectorSubcoreMesh(
    core_axis_name="c", subcore_axis_name="s", num_cores=NC, num_subcores=NS
)
_SC_PARAMS = pltpu.CompilerParams(use_tc_tiling_on_sc=False)


DEG_CHUNKS = CHUNKS // NC


@functools.partial(
    pl.kernel,
    out_type=jax.ShapeDtypeStruct((NC * NPAD,), jnp.float32),
    mesh=_MESH,
    compiler_params=_SC_PARAMS,
    scratch_types=[
        pltpu.VMEM((DEG_CHUNKS, B), jnp.int32),
        pltpu.VMEM((B,), jnp.float32),
        pltpu.VMEM((640,), jnp.float32),
        pltpu.VMEM_SHARED((NPAD,), jnp.float32),
        pltpu.SemaphoreType.DMA,
    ],
)
def _sc_degree(er_hbm, out_hbm, dstb, onesv, zb, acc, dsem):
    c = lax.axis_index("c")
    s = lax.axis_index("s")

    pltpu.sync_copy(er_hbm.at[1, s, pl.ds(c * DEG_CHUNKS, DEG_CHUNKS)], dstb)
    for j in range(640 // 16):
        zb[pl.ds(j * 16, 16)] = jnp.zeros((16,), jnp.float32)
    for j in range(B // 16):
        onesv[pl.ds(j * 16, 16)] = jnp.ones((16,), jnp.float32)

    pltpu.sync_copy(zb.at[pl.ds(0, RPT)], acc.at[pl.ds(s * RPT, RPT)])
    plsc.subcore_barrier()

    def body(i, carry):
        pltpu.async_copy(onesv, acc.at[dstb.at[i]], dsem, add=True)
        return carry

    lax.fori_loop(0, DEG_CHUNKS, body, 0)

    def drain(i, carry):
        pltpu.make_async_copy(onesv, acc.at[dstb.at[i]], dsem).wait()
        return carry

    lax.fori_loop(0, DEG_CHUNKS, drain, 0)
    plsc.subcore_barrier()
    pltpu.sync_copy(acc.at[pl.ds(s * RPT, RPT)], zb.at[pl.ds(0, RPT)])
    pltpu.sync_copy(zb.at[pl.ds(0, RPT)], out_hbm.at[pl.ds(c * NPAD + s * RPT, RPT)])


NBUF = 5
PFD = 3
OUTCH = 4


def _make_prop(fwh):
    @functools.partial(
        pl.kernel,
        out_type=jax.ShapeDtypeStruct((NC, NPAD, fwh), jnp.float32),
        mesh=_MESH,
        compiler_params=_SC_PARAMS,
        scratch_types=[
            pltpu.VMEM((CHUNKS, B), jnp.int32),
            pltpu.VMEM((CHUNKS, B), jnp.int32),
            [pltpu.VMEM((B, fwh), jnp.float32) for _ in range(NBUF)],
            pltpu.VMEM((RPT // OUTCH, fwh), jnp.float32),
            pltpu.VMEM_SHARED((NPAD, fwh), jnp.float32),
            [pltpu.SemaphoreType.DMA for _ in range(NBUF)],
            [pltpu.SemaphoreType.DMA for _ in range(NBUF)],
            pltpu.SemaphoreType.DMA,
        ],
    )
    def _prop(er_hbm, g_hbm, out_hbm, srcb, dstb, rows, stg, acc,
              gsem, ssem, zsem):
        c = lax.axis_index("c")
        s = lax.axis_index("s")

        pltpu.sync_copy(er_hbm.at[0, s], srcb)
        pltpu.sync_copy(er_hbm.at[1, s], dstb)

        for r in range(8):
            for j in range(fwh // 16):
                stg[r, pl.ds(j * 16, 16)] = jnp.zeros((16,), jnp.float32)

        def zinit(i, carry):
            pltpu.async_copy(
                stg.at[pl.ds(0, 8)], acc.at[pl.ds(s * RPT + i * 8, 8)], zsem
            )
            return carry

        lax.fori_loop(0, RPT // 8, zinit, 0)
        pltpu.make_async_copy(
            out_hbm.at[c, pl.ds(s * RPT, RPT)], acc.at[pl.ds(s * RPT, RPT)], zsem
        ).wait()
        plsc.subcore_barrier()

        gtab = g_hbm.at[c]

        def _gather_start(i, k):
            pltpu.async_copy(gtab.at[srcb.at[i]], rows[k], gsem[k])

        def _gather_wait(i, k):
            pltpu.make_async_copy(gtab.at[srcb.at[i]], rows[k], gsem[k]).wait()

        def _scat_start(i, k):
            pltpu.async_copy(rows[k], acc.at[dstb.at[i]], ssem[k], add=True)

        def _scat_wait(i, k):
            pltpu.make_async_copy(rows[k], acc.at[dstb.at[i]], ssem[k]).wait()

        for k in range(PFD):
            _gather_start(k, k)

        def body(jj, carry):
            for k in range(NBUF):
                i = jj * NBUF + k
                _gather_wait(i, k)
                _scat_start(i, k)
                kp = (k + PFD) % NBUF
                ip = i + PFD
                if k + PFD < NBUF:
                    @pl.when(jj > 0)
                    def _():
                        _scat_wait(ip - NBUF, kp)
                        _gather_start(ip, kp)

                    @pl.when(jj == 0)
                    def _():
                        _gather_start(ip, kp)
                else:

                    @pl.when(ip < CHUNKS)
                    def _():
                        _scat_wait(ip - NBUF, kp)
                        _gather_start(ip, kp)

            return carry

        lax.fori_loop(0, CHUNKS // NBUF, body, 0)
        for k in range(NBUF):
            i = CHUNKS - NBUF + k
            _scat_wait(i, k)

        plsc.subcore_barrier()
        ocr = RPT // OUTCH
        for j in range(OUTCH):
            r0 = s * RPT + j * ocr
            pltpu.sync_copy(acc.at[pl.ds(r0, ocr)], stg)
            pltpu.sync_copy(stg, out_hbm.at[c, pl.ds(r0, ocr)])

    return _prop


_prop_h = _make_prop(H // 2)
_prop_c = _make_prop(CP // 2)


RB = 2000
GRID = N // RB


def _tc1_body(degt_ref, x_ref, w_ref, dinv_ref, g_ref):
    deg = degt_ref[:, 0:1] + degt_ref[:, 1:2] + 1.0
    dinv = lax.rsqrt(deg)
    dinv_ref[...] = dinv
    z = dinv * jnp.dot(x_ref[...], w_ref[...], preferred_element_type=jnp.float32)
    g_ref[0] = z[:, : H // 2]
    g_ref[1] = z[:, H // 2 :]


def _tc1(degt, x, w1):
    return pl.pallas_call(
        _tc1_body,
        grid=(GRID,),
        in_specs=[
            pl.BlockSpec((RB, 2), lambda i: (i, 0)),
            pl.BlockSpec((RB, F), lambda i: (i, 0)),
            pl.BlockSpec((F, H), lambda i: (0, 0)),
        ],
        out_specs=[
            pl.BlockSpec((RB, 1), lambda i: (i, 0)),
            pl.BlockSpec((NC, RB, H // 2), lambda i: (0, i, 0)),
        ],
        out_shape=[
            jax.ShapeDtypeStruct((N, 1), jnp.float32),
            jax.ShapeDtypeStruct((NC, N, H // 2), jnp.float32),
        ],
    )(degt, x, w1)


def _tc_mid_body(s_ref, g_ref, dinv_ref, b_ref, w_ref, out_ref):
    fo2 = out_ref.shape[2]
    dinv = dinv_ref[...]
    sg = jnp.concatenate([s_ref[0] + g_ref[0], s_ref[1] + g_ref[1]], axis=1)
    h = jnp.maximum(dinv * sg + b_ref[...], 0.0)
    z = dinv * jnp.dot(h, w_ref[...], preferred_element_type=jnp.float32)
    out_ref[0] = z[:, :fo2]
    out_ref[1] = z[:, fo2:]


def _tc_mid(s, g, dinv, b, w, fout):
    fin = 2 * g.shape[2]
    return pl.pallas_call(
        _tc_mid_body,
        grid=(GRID,),
        in_specs=[
            pl.BlockSpec((NC, RB, fin // 2), lambda i: (0, i, 0)),
            pl.BlockSpec((NC, RB, fin // 2), lambda i: (0, i, 0)),
            pl.BlockSpec((RB, 1), lambda i: (i, 0)),
            pl.BlockSpec((1, fin), lambda i: (0, 0)),
            pl.BlockSpec((fin, fout), lambda i: (0, 0)),
        ],
        out_specs=pl.BlockSpec((NC, RB, fout // 2), lambda i: (0, i, 0)),
        out_shape=jax.ShapeDtypeStruct((NC, N, fout // 2), jnp.float32),
    )(s, g, dinv, b, w)


def _tc_out_body(s_ref, g_ref, dinv_ref, b_ref, out_ref):
    sg = jnp.concatenate([s_ref[0] + g_ref[0], s_ref[1] + g_ref[1]], axis=1)
    o = dinv_ref[...] * sg + b_ref[...]
    col = lax.broadcasted_iota(jnp.int32, (RB, CP), 1)
    o = jnp.where(col < C, o, -jnp.inf)
    m = jnp.max(o, axis=1, keepdims=True)
    e = o - m
    lse = jnp.log(jnp.sum(jnp.exp(e), axis=1, keepdims=True))
    out_ref[...] = e - lse


def _tc_out(s, g, dinv, b):
    return pl.pallas_call(
        _tc_out_body,
        grid=(GRID,),
        in_specs=[
            pl.BlockSpec((NC, RB, CP // 2), lambda i: (0, i, 0)),
            pl.BlockSpec((NC, RB, CP // 2), lambda i: (0, i, 0)),
            pl.BlockSpec((RB, 1), lambda i: (i, 0)),
            pl.BlockSpec((1, CP), lambda i: (0, 0)),
        ],
        out_specs=pl.BlockSpec((RB, CP), lambda i: (i, 0)),
        out_shape=jax.ShapeDtypeStruct((N, CP), jnp.float32),
    )(s, g, dinv, b)


def kernel(x, edge_index, W1, b1, W2, b2, W3, b3):
    er = edge_index.reshape(2, NS, CHUNKS, B)

    deg = _sc_degree(er).reshape(NC, NPAD)
    degt = jnp.transpose(deg)[:N]

    dinv, g1 = _tc1(degt, x, W1)
    s1 = _prop_h(er, g1)
    g2 = _tc_mid(s1, g1, dinv, b1.reshape(1, H), W2, H)
    s2 = _prop_h(er, g2)
    w3p = jnp.pad(W3, ((0, 0), (0, CP - C)))
    b3p = jnp.pad(b3, (0, CP - C))
    g3 = _tc_mid(s2, g2, dinv, b2.reshape(1, H), w3p, CP)
    s3 = _prop_c(er, g3)
    out = _tc_out(s3, g3, dinv, b3p.reshape(1, CP))
    return out[:, :C]


# speedup vs baseline: 2.0024x; 1.0674x over previous
import functools

import jax
import jax.numpy as jnp
from jax import lax
from jax.experimental import pallas as pl
from jax.experimental.pallas import tpu as pltpu
from jax.experimental.pallas import tpu_sc as plsc

N = 10000
E = 320000
F = 128
H = 128
C = 40
CP = 48

NC = 2
NS = 16

B = 80
CHUNKS = 250
EPT = CHUNKS * B
NPAD = 10112
RPT = NPAD // NS

_MESH = plsc.VectorSubcoreMesh(
    core_axis_name="c", subcore_axis_name="s", num_cores=NC, num_subcores=NS
)
_SC_PARAMS = pltpu.CompilerParams(use_tc_tiling_on_sc=False)


DEG_CHUNKS = CHUNKS // NC


@functools.partial(
    pl.kernel,
    out_type=jax.ShapeDtypeStruct((NC * NPAD,), jnp.float32),
    mesh=_MESH,
    compiler_params=_SC_PARAMS,
    scratch_types=[
        pltpu.VMEM((DEG_CHUNKS, B), jnp.int32),
        pltpu.VMEM((B,), jnp.float32),
        pltpu.VMEM((640,), jnp.float32),
        pltpu.VMEM_SHARED((NPAD,), jnp.float32),
        pltpu.SemaphoreType.DMA,
    ],
)
def _sc_degree(er_hbm, out_hbm, dstb, onesv, zb, acc, dsem):
    c = lax.axis_index("c")
    s = lax.axis_index("s")

    pltpu.sync_copy(er_hbm.at[1, s, pl.ds(c * DEG_CHUNKS, DEG_CHUNKS)], dstb)
    for j in range(640 // 16):
        zb[pl.ds(j * 16, 16)] = jnp.zeros((16,), jnp.float32)
    for j in range(B // 16):
        onesv[pl.ds(j * 16, 16)] = jnp.ones((16,), jnp.float32)

    pltpu.sync_copy(zb.at[pl.ds(0, RPT)], acc.at[pl.ds(s * RPT, RPT)])
    plsc.subcore_barrier()

    def body(i, carry):
        pltpu.async_copy(onesv, acc.at[dstb.at[i]], dsem, add=True)
        return carry

    lax.fori_loop(0, DEG_CHUNKS, body, 0)

    def drain(i, carry):
        pltpu.make_async_copy(onesv, acc.at[dstb.at[i]], dsem).wait()
        return carry

    lax.fori_loop(0, DEG_CHUNKS, drain, 0)
    plsc.subcore_barrier()
    pltpu.sync_copy(acc.at[pl.ds(s * RPT, RPT)], zb.at[pl.ds(0, RPT)])
    pltpu.sync_copy(zb.at[pl.ds(0, RPT)], out_hbm.at[pl.ds(c * NPAD + s * RPT, RPT)])


NBUF = 5
PFD = 3
OUTCH = 4


def _make_prop(fwh):
    @functools.partial(
        pl.kernel,
        out_type=jax.ShapeDtypeStruct((NPAD, 2 * fwh), jnp.float32),
        mesh=_MESH,
        compiler_params=_SC_PARAMS,
        scratch_types=[
            pltpu.VMEM((CHUNKS, B), jnp.int32),
            pltpu.VMEM((CHUNKS, B), jnp.int32),
            [pltpu.VMEM((B, fwh), jnp.float32) for _ in range(NBUF)],
            pltpu.VMEM((RPT // OUTCH, fwh), jnp.float32),
            pltpu.VMEM_SHARED((NPAD, fwh), jnp.float32),
            [pltpu.SemaphoreType.DMA for _ in range(NBUF)],
            [pltpu.SemaphoreType.DMA for _ in range(NBUF)],
            pltpu.SemaphoreType.DMA,
        ],
    )
    def _prop(er_hbm, g_hbm, out_hbm, srcb, dstb, rows, stg, acc,
              gsem, ssem, zsem):
        c = lax.axis_index("c")
        s = lax.axis_index("s")

        pltpu.sync_copy(er_hbm.at[0, s], srcb)
        pltpu.sync_copy(er_hbm.at[1, s], dstb)

        for r in range(8):
            for j in range(fwh // 16):
                stg[r, pl.ds(j * 16, 16)] = jnp.zeros((16,), jnp.float32)

        def zinit(i, carry):
            pltpu.async_copy(
                stg.at[pl.ds(0, 8)], acc.at[pl.ds(s * RPT + i * 8, 8)], zsem
            )
            return carry

        lax.fori_loop(0, RPT // 8, zinit, 0)
        pltpu.make_async_copy(
            out_hbm.at[pl.ds(s * RPT, RPT), pl.ds(c * fwh, fwh)],
            acc.at[pl.ds(s * RPT, RPT)],
            zsem,
        ).wait()
        plsc.subcore_barrier()

        gtab = g_hbm.at[c]

        def _gather_start(i, k):
            pltpu.async_copy(gtab.at[srcb.at[i]], rows[k], gsem[k])

        def _gather_wait(i, k):
            pltpu.make_async_copy(gtab.at[srcb.at[i]], rows[k], gsem[k]).wait()

        def _scat_start(i, k):
            pltpu.async_copy(rows[k], acc.at[dstb.at[i]], ssem[k], add=True)

        def _scat_wait(i, k):
            pltpu.make_async_copy(rows[k], acc.at[dstb.at[i]], ssem[k]).wait()

        for k in range(PFD):
            _gather_start(k, k)

        def body(jj, carry):
            for k in range(NBUF):
                i = jj * NBUF + k
                _gather_wait(i, k)
                _scat_start(i, k)
                kp = (k + PFD) % NBUF
                ip = i + PFD
                if k + PFD < NBUF:
                    @pl.when(jj > 0)
                    def _():
                        _scat_wait(ip - NBUF, kp)
                        _gather_start(ip, kp)

                    @pl.when(jj == 0)
                    def _():
                        _gather_start(ip, kp)
                else:

                    @pl.when(ip < CHUNKS)
                    def _():
                        _scat_wait(ip - NBUF, kp)
                        _gather_start(ip, kp)

            return carry

        lax.fori_loop(0, CHUNKS // NBUF, body, 0)
        for k in range(NBUF):
            i = CHUNKS - NBUF + k
            _scat_wait(i, k)

        plsc.subcore_barrier()
        ocr = RPT // OUTCH
        for j in range(OUTCH):
            r0 = s * RPT + j * ocr
            pltpu.sync_copy(acc.at[pl.ds(r0, ocr)], stg)
            pltpu.sync_copy(
                stg, out_hbm.at[pl.ds(r0, ocr), pl.ds(c * fwh, fwh)]
            )

    return _prop


_prop_h = _make_prop(H // 2)
_prop_c = _make_prop(CP // 2)


RB = 2000
GRID = N // RB


def _tc1_body(degt_ref, x_ref, w_ref, dinv_ref, g_ref):
    deg = degt_ref[:, 0:1] + degt_ref[:, 1:2] + 1.0
    dinv = lax.rsqrt(deg)
    dinv_ref[...] = dinv
    z = dinv * jnp.dot(x_ref[...], w_ref[...], preferred_element_type=jnp.float32)
    g_ref[0] = z[:, : H // 2]
    g_ref[1] = z[:, H // 2 :]


def _tc1(degt, x, w1):
    return pl.pallas_call(
        _tc1_body,
        grid=(GRID,),
        in_specs=[
            pl.BlockSpec((RB, 2), lambda i: (i, 0)),
            pl.BlockSpec((RB, F), lambda i: (i, 0)),
            pl.BlockSpec((F, H), lambda i: (0, 0)),
        ],
        out_specs=[
            pl.BlockSpec((RB, 1), lambda i: (i, 0)),
            pl.BlockSpec((NC, RB, H // 2), lambda i: (0, i, 0)),
        ],
        out_shape=[
            jax.ShapeDtypeStruct((N, 1), jnp.float32),
            jax.ShapeDtypeStruct((NC, N, H // 2), jnp.float32),
        ],
    )(degt, x, w1)


def _tc_mid_body(s_ref, g_ref, dinv_ref, b_ref, w_ref, out_ref):
    fo2 = out_ref.shape[2]
    dinv = dinv_ref[...]
    g = jnp.concatenate([g_ref[0], g_ref[1]], axis=1)
    h = jnp.maximum(dinv * (s_ref[...] + g) + b_ref[...], 0.0)
    z = dinv * jnp.dot(h, w_ref[...], preferred_element_type=jnp.float32)
    out_ref[0] = z[:, :fo2]
    out_ref[1] = z[:, fo2:]


def _tc_mid(s, g, dinv, b, w, fout):
    fin = 2 * g.shape[2]
    return pl.pallas_call(
        _tc_mid_body,
        grid=(GRID,),
        in_specs=[
            pl.BlockSpec((RB, fin), lambda i: (i, 0)),
            pl.BlockSpec((NC, RB, fin // 2), lambda i: (0, i, 0)),
            pl.BlockSpec((RB, 1), lambda i: (i, 0)),
            pl.BlockSpec((1, fin), lambda i: (0, 0)),
            pl.BlockSpec((fin, fout), lambda i: (0, 0)),
        ],
        out_specs=pl.BlockSpec((NC, RB, fout // 2), lambda i: (0, i, 0)),
        out_shape=jax.ShapeDtypeStruct((NC, N, fout // 2), jnp.float32),
    )(s, g, dinv, b, w)


def _tc_out_body(s_ref, g_ref, dinv_ref, b_ref, out_ref):
    g = jnp.concatenate([g_ref[0], g_ref[1]], axis=1)
    o = dinv_ref[...] * (s_ref[...] + g) + b_ref[...]
    col = lax.broadcasted_iota(jnp.int32, (RB, CP), 1)
    o = jnp.where(col < C, o, -jnp.inf)
    m = jnp.max(o, axis=1, keepdims=True)
    e = o - m
    lse = jnp.log(jnp.sum(jnp.exp(e), axis=1, keepdims=True))
    out_ref[...] = e - lse


def _tc_out(s, g, dinv, b):
    return pl.pallas_call(
        _tc_out_body,
        grid=(GRID,),
        in_specs=[
            pl.BlockSpec((RB, CP), lambda i: (i, 0)),
            pl.BlockSpec((NC, RB, CP // 2), lambda i: (0, i, 0)),
            pl.BlockSpec((RB, 1), lambda i: (i, 0)),
            pl.BlockSpec((1, CP), lambda i: (0, 0)),
        ],
        out_specs=pl.BlockSpec((RB, CP), lambda i: (i, 0)),
        out_shape=jax.ShapeDtypeStruct((N, CP), jnp.float32),
    )(s, g, dinv, b)


def kernel(x, edge_index, W1, b1, W2, b2, W3, b3):
    er = edge_index.reshape(2, NS, CHUNKS, B)

    deg = _sc_degree(er).reshape(NC, NPAD)
    degt = jnp.transpose(deg)[:N]

    dinv, g1 = _tc1(degt, x, W1)
    s1 = _prop_h(er, g1)
    g2 = _tc_mid(s1, g1, dinv, b1.reshape(1, H), W2, H)
    s2 = _prop_h(er, g2)
    w3p = jnp.pad(W3, ((0, 0), (0, CP - C)))
    b3p = jnp.pad(b3, (0, CP - C))
    g3 = _tc_mid(s2, g2, dinv, b2.reshape(1, H), w3p, CP)
    s3 = _prop_c(er, g3)
    out = _tc_out(s3, g3, dinv, b3p.reshape(1, CP))
    return out[:, :C]


# speedup vs baseline: 2.1789x; 1.0881x over previous
import functools

import jax
import jax.numpy as jnp
from jax import lax
from jax.experimental import pallas as pl
from jax.experimental.pallas import tpu as pltpu
from jax.experimental.pallas import tpu_sc as plsc

N = 10000
E = 320000
F = 128
H = 128
C = 40
CP = 48

NC = 2
NS = 16

B = 80
CHUNKS = 250
EPT = CHUNKS * B
NPAD = 10112
RPT = NPAD // NS

_MESH = plsc.VectorSubcoreMesh(
    core_axis_name="c", subcore_axis_name="s", num_cores=NC, num_subcores=NS
)
_SC_PARAMS = pltpu.CompilerParams(use_tc_tiling_on_sc=False)


DEG_CHUNKS = CHUNKS // NC


@functools.partial(
    pl.kernel,
    out_type=jax.ShapeDtypeStruct((NC * NPAD,), jnp.float32),
    mesh=_MESH,
    compiler_params=_SC_PARAMS,
    scratch_types=[
        pltpu.VMEM((DEG_CHUNKS, B), jnp.int32),
        pltpu.VMEM((B,), jnp.float32),
        pltpu.VMEM((640,), jnp.float32),
        pltpu.VMEM_SHARED((NPAD,), jnp.float32),
        pltpu.SemaphoreType.DMA,
    ],
)
def _sc_degree(er_hbm, out_hbm, dstb, onesv, zb, acc, dsem):
    c = lax.axis_index("c")
    s = lax.axis_index("s")

    pltpu.sync_copy(er_hbm.at[1, s, pl.ds(c * DEG_CHUNKS, DEG_CHUNKS)], dstb)
    for j in range(640 // 16):
        zb[pl.ds(j * 16, 16)] = jnp.zeros((16,), jnp.float32)
    for j in range(B // 16):
        onesv[pl.ds(j * 16, 16)] = jnp.ones((16,), jnp.float32)

    pltpu.sync_copy(zb.at[pl.ds(0, RPT)], acc.at[pl.ds(s * RPT, RPT)])
    plsc.subcore_barrier()

    def body(i, carry):
        pltpu.async_copy(onesv, acc.at[dstb.at[i]], dsem, add=True)
        return carry

    lax.fori_loop(0, DEG_CHUNKS, body, 0)

    def drain(i, carry):
        pltpu.make_async_copy(onesv, acc.at[dstb.at[i]], dsem).wait()
        return carry

    lax.fori_loop(0, DEG_CHUNKS, drain, 0)
    plsc.subcore_barrier()
    pltpu.sync_copy(acc.at[pl.ds(s * RPT, RPT)], zb.at[pl.ds(0, RPT)])
    pltpu.sync_copy(zb.at[pl.ds(0, RPT)], out_hbm.at[pl.ds(c * NPAD + s * RPT, RPT)])


NBUF = 5
PFD = 4
OUTCH = 4


def _make_prop(fwh):
    @functools.partial(
        pl.kernel,
        out_type=jax.ShapeDtypeStruct((NPAD, 2 * fwh), jnp.float32),
        mesh=_MESH,
        compiler_params=_SC_PARAMS,
        scratch_types=[
            pltpu.VMEM((CHUNKS, B), jnp.int32),
            pltpu.VMEM((CHUNKS, B), jnp.int32),
            [pltpu.VMEM((B, fwh), jnp.float32) for _ in range(NBUF)],
            pltpu.VMEM((RPT // OUTCH, fwh), jnp.float32),
            pltpu.VMEM_SHARED((NPAD, fwh), jnp.float32),
            [pltpu.SemaphoreType.DMA for _ in range(NBUF)],
            [pltpu.SemaphoreType.DMA for _ in range(NBUF)],
            pltpu.SemaphoreType.DMA,
        ],
    )
    def _prop(er_hbm, g_hbm, out_hbm, srcb, dstb, rows, stg, acc,
              gsem, ssem, zsem):
        c = lax.axis_index("c")
        s = lax.axis_index("s")

        pltpu.sync_copy(er_hbm.at[0, s], srcb)
        pltpu.sync_copy(er_hbm.at[1, s], dstb)

        for r in range(8):
            for j in range(fwh // 16):
                stg[r, pl.ds(j * 16, 16)] = jnp.zeros((16,), jnp.float32)

        def zinit(i, carry):
            pltpu.async_copy(
                stg.at[pl.ds(0, 8)], acc.at[pl.ds(s * RPT + i * 8, 8)], zsem
            )
            return carry

        lax.fori_loop(0, RPT // 8, zinit, 0)
        pltpu.make_async_copy(
            out_hbm.at[pl.ds(s * RPT, RPT), pl.ds(c * fwh, fwh)],
            acc.at[pl.ds(s * RPT, RPT)],
            zsem,
        ).wait()
        plsc.subcore_barrier()

        gtab = g_hbm.at[c]

        def _gather_start(i, k):
            pltpu.async_copy(gtab.at[srcb.at[i]], rows[k], gsem[k])

        def _gather_wait(i, k):
            pltpu.make_async_copy(gtab.at[srcb.at[i]], rows[k], gsem[k]).wait()

        def _scat_start(i, k):
            pltpu.async_copy(rows[k], acc.at[dstb.at[i]], ssem[k], add=True)

        def _scat_wait(i, k):
            pltpu.make_async_copy(rows[k], acc.at[dstb.at[i]], ssem[k]).wait()

        for k in range(PFD):
            _gather_start(k, k)

        def body(jj, carry):
            for k in range(NBUF):
                i = jj * NBUF + k
                _gather_wait(i, k)
                _scat_start(i, k)
                kp = (k + PFD) % NBUF
                ip = i + PFD
                if k + PFD < NBUF:
                    @pl.when(jj > 0)
                    def _():
                        _scat_wait(ip - NBUF, kp)
                        _gather_start(ip, kp)

                    @pl.when(jj == 0)
                    def _():
                        _gather_start(ip, kp)
                else:

                    @pl.when(ip < CHUNKS)
                    def _():
                        _scat_wait(ip - NBUF, kp)
                        _gather_start(ip, kp)

            return carry

        lax.fori_loop(0, CHUNKS // NBUF, body, 0)
        for k in range(NBUF):
            i = CHUNKS - NBUF + k
            _scat_wait(i, k)

        plsc.subcore_barrier()
        ocr = RPT // OUTCH
        for j in range(OUTCH):
            r0 = s * RPT + j * ocr
            pltpu.sync_copy(acc.at[pl.ds(r0, ocr)], stg)
            pltpu.sync_copy(
                stg, out_hbm.at[pl.ds(r0, ocr), pl.ds(c * fwh, fwh)]
            )

    return _prop


_prop_h = _make_prop(H // 2)
_prop_c = _make_prop(CP // 2)


RB = 2000
GRID = N // RB


def _tc1_body(degt_ref, x_ref, w_ref, dinv_ref, g_ref):
    deg = degt_ref[:, 0:1] + degt_ref[:, 1:2] + 1.0
    dinv = lax.rsqrt(deg)
    dinv_ref[...] = dinv
    z = dinv * jnp.dot(x_ref[...], w_ref[...], preferred_element_type=jnp.float32)
    g_ref[0] = z[:, : H // 2]
    g_ref[1] = z[:, H // 2 :]


def _tc1(degt, x, w1):
    return pl.pallas_call(
        _tc1_body,
        grid=(GRID,),
        in_specs=[
            pl.BlockSpec((RB, 2), lambda i: (i, 0)),
            pl.BlockSpec((RB, F), lambda i: (i, 0)),
            pl.BlockSpec((F, H), lambda i: (0, 0)),
        ],
        out_specs=[
            pl.BlockSpec((RB, 1), lambda i: (i, 0)),
            pl.BlockSpec((NC, RB, H // 2), lambda i: (0, i, 0)),
        ],
        out_shape=[
            jax.ShapeDtypeStruct((N, 1), jnp.float32),
            jax.ShapeDtypeStruct((NC, N, H // 2), jnp.float32),
        ],
    )(degt, x, w1)


def _tc_mid_body(s_ref, g_ref, dinv_ref, b_ref, w_ref, out_ref):
    fo2 = out_ref.shape[2]
    dinv = dinv_ref[...]
    g = jnp.concatenate([g_ref[0], g_ref[1]], axis=1)
    h = jnp.maximum(dinv * (s_ref[...] + g) + b_ref[...], 0.0)
    z = dinv * jnp.dot(h, w_ref[...], preferred_element_type=jnp.float32)
    out_ref[0] = z[:, :fo2]
    out_ref[1] = z[:, fo2:]


def _tc_mid(s, g, dinv, b, w, fout):
    fin = 2 * g.shape[2]
    return pl.pallas_call(
        _tc_mid_body,
        grid=(GRID,),
        in_specs=[
            pl.BlockSpec((RB, fin), lambda i: (i, 0)),
            pl.BlockSpec((NC, RB, fin // 2), lambda i: (0, i, 0)),
            pl.BlockSpec((RB, 1), lambda i: (i, 0)),
            pl.BlockSpec((1, fin), lambda i: (0, 0)),
            pl.BlockSpec((fin, fout), lambda i: (0, 0)),
        ],
        out_specs=pl.BlockSpec((NC, RB, fout // 2), lambda i: (0, i, 0)),
        out_shape=jax.ShapeDtypeStruct((NC, N, fout // 2), jnp.float32),
    )(s, g, dinv, b, w)


def _tc_out_body(s_ref, g_ref, dinv_ref, b_ref, out_ref):
    g = jnp.concatenate([g_ref[0], g_ref[1]], axis=1)
    o = dinv_ref[...] * (s_ref[...] + g) + b_ref[...]
    col = lax.broadcasted_iota(jnp.int32, (RB, CP), 1)
    o = jnp.where(col < C, o, -jnp.inf)
    m = jnp.max(o, axis=1, keepdims=True)
    e = o - m
    lse = jnp.log(jnp.sum(jnp.exp(e), axis=1, keepdims=True))
    out_ref[...] = (e - lse)[:, :C]


def _tc_out(s, g, dinv, b):
    return pl.pallas_call(
        _tc_out_body,
        grid=(GRID,),
        in_specs=[
            pl.BlockSpec((RB, CP), lambda i: (i, 0)),
            pl.BlockSpec((NC, RB, CP // 2), lambda i: (0, i, 0)),
            pl.BlockSpec((RB, 1), lambda i: (i, 0)),
            pl.BlockSpec((1, CP), lambda i: (0, 0)),
        ],
        out_specs=pl.BlockSpec((RB, C), lambda i: (i, 0)),
        out_shape=jax.ShapeDtypeStruct((N, C), jnp.float32),
    )(s, g, dinv, b)


def kernel(x, edge_index, W1, b1, W2, b2, W3, b3):
    er = edge_index.reshape(2, NS, CHUNKS, B)

    deg = _sc_degree(er).reshape(NC, NPAD)
    degt = jnp.transpose(deg)[:N]

    dinv, g1 = _tc1(degt, x, W1)
    s1 = _prop_h(er, g1)
    g2 = _tc_mid(s1, g1, dinv, b1.reshape(1, H), W2, H)
    s2 = _prop_h(er, g2)
    w3p = jnp.pad(W3, ((0, 0), (0, CP - C)))
    b3p = jnp.pad(b3, (0, CP - C))
    g3 = _tc_mid(s2, g2, dinv, b2.reshape(1, H), w3p, CP)
    s3 = _prop_c(er, g3)
    return _tc_out(s3, g3, dinv, b3p.reshape(1, CP))
